# Initial kernel scaffold; baseline (speedup 1.0000x reference)
import jax
import jax.numpy as jnp
from jax.experimental import pallas as pl


def kernel(hidden_states, position_ids, Wq, Wk, Wv, Wo):
    raise NotImplementedError("write your pallas kernel here")


# speedup vs baseline: 20.7053x; 20.7053x over previous
import functools

import jax
import jax.numpy as jnp
import numpy as np
from jax.experimental import pallas as pl

B = 1
S = 2048
D = 1024
H = 16
DH = D // H
SB = 256
NSB = S // SB
NEG = float(np.finfo(np.float32).min)
INT_MIN = np.int32(-2**31)
MASK30 = np.int32(0x7FFFFFFF)

_PREC = jax.lax.Precision.DEFAULT


def _dot(a, b, dims):
    return jax.lax.dot_general(a, b, (dims, ((), ())),
                               preferred_element_type=jnp.float32,
                               precision=_PREC)


def _qkv_kernel(hid_ref, wq_ref, wk_ref, wv_ref, cs_ref, q_ref, k_ref, v_ref):
    h = hid_ref[...]
    cos = cs_ref[0]
    sin = cs_ref[1]

    def proj(w_ref, rope):
        x = _dot(h, w_ref[0], ((1,), (0,)))
        if rope:
            rot = jnp.concatenate([-x[:, DH // 2:], x[:, :DH // 2]], axis=1)
            x = x * cos + rot * sin
        return x

    q_ref[0] = proj(wq_ref, True)
    k_ref[0] = proj(wk_ref, True)
    v_ref[0] = proj(wv_ref, False)


def _attn_kernel(q_ref, k_ref, v_ref, o_ref):
    qb = pl.program_id(1)
    q = q_ref[0]
    k = k_ref[0]
    v = v_ref[0]

    s = _dot(q, k, ((1,), (1,))) * (1.0 / float(np.sqrt(DH)))

    row = qb * SB + jax.lax.broadcasted_iota(jnp.int32, (SB, 1), 0)
    col = jax.lax.broadcasted_iota(jnp.int32, (SB, S), 1)
    causal = col <= row
    kk = jnp.maximum(row // 4 - 3, 0)
    kf = kk.astype(jnp.float32)

    bits = jax.lax.bitcast_convert_type(s, jnp.int32)
    mono = jnp.where(bits < 0, bits ^ MASK30, bits)
    mono = jnp.where(causal, mono, INT_MIN)

    c0 = jnp.sum(jnp.where(mono >= 0, 1.0, 0.0), axis=1, keepdims=True)
    base = jnp.where(c0 >= kf, jnp.int32(0), INT_MIN)

    def body(i, base):
        cand = base + jax.lax.shift_left(jnp.int32(1), 30 - i)
        c = jnp.sum(jnp.where(mono >= cand, 1.0, 0.0), axis=1, keepdims=True)
        return jnp.where(c >= kf, cand, base)

    base = jax.lax.fori_loop(0, 31, body, base)

    allowed = causal & (((mono >= base) & (kk > 0)) | (col < 4))
    sm = jnp.where(allowed, s, NEG)
    m = jnp.max(sm, axis=1, keepdims=True)
    e = jnp.where(allowed, jnp.exp(s - m), 0.0)
    p = e / jnp.sum(e, axis=1, keepdims=True)

    o_ref[0] = _dot(p, v, ((1,), (0,)))


def _out_kernel(x_ref, wo_ref, o_ref):
    acc = _dot(x_ref[0], wo_ref[0], ((1,), (0,)))
    for h in range(1, H):
        acc = acc + _dot(x_ref[h], wo_ref[h], ((1,), (0,)))
    o_ref[...] = acc


@jax.jit
def kernel(hidden_states, position_ids, Wq, Wk, Wv, Wo):
    hid = hidden_states[0]

    inv_freq = 1.0 / (10000.0 ** (jnp.arange(0, DH, 2, dtype=jnp.float32) / DH))
    t = position_ids[0].astype(jnp.float32)
    freqs = t[:, None] * inv_freq[None, :]
    emb = jnp.concatenate([freqs, freqs], axis=-1)
    cs = jnp.stack([jnp.cos(emb), jnp.sin(emb)])

    wqT = Wq.T.reshape(D, H, DH).transpose(1, 0, 2)
    wkT = Wk.T.reshape(D, H, DH).transpose(1, 0, 2)
    wvT = Wv.T.reshape(D, H, DH).transpose(1, 0, 2)
    woT = Wo.T.reshape(H, DH, D)

    q, k, v = pl.pallas_call(
        _qkv_kernel,
        grid=(NSB, H),
        in_specs=[
            pl.BlockSpec((SB, D), lambda i, h: (i, 0)),
            pl.BlockSpec((1, D, DH), lambda i, h: (h, 0, 0)),
            pl.BlockSpec((1, D, DH), lambda i, h: (h, 0, 0)),
            pl.BlockSpec((1, D, DH), lambda i, h: (h, 0, 0)),
            pl.BlockSpec((2, SB, DH), lambda i, h: (0, i, 0)),
        ],
        out_specs=[
            pl.BlockSpec((1, SB, DH), lambda i, h: (h, i, 0)),
            pl.BlockSpec((1, SB, DH), lambda i, h: (h, i, 0)),
            pl.BlockSpec((1, SB, DH), lambda i, h: (h, i, 0)),
        ],
        out_shape=[jax.ShapeDtypeStruct((H, S, DH), jnp.float32)] * 3,
    )(hid, wqT, wkT, wvT, cs)

    attn_out = pl.pallas_call(
        _attn_kernel,
        grid=(H, NSB),
        in_specs=[
            pl.BlockSpec((1, SB, DH), lambda h, i: (h, i, 0)),
            pl.BlockSpec((1, S, DH), lambda h, i: (h, 0, 0)),
            pl.BlockSpec((1, S, DH), lambda h, i: (h, 0, 0)),
        ],
        out_specs=pl.BlockSpec((1, SB, DH), lambda h, i: (h, i, 0)),
        out_shape=jax.ShapeDtypeStruct((H, S, DH), jnp.float32),
    )(q, k, v)

    out = pl.pallas_call(
        _out_kernel,
        grid=(NSB,),
        in_specs=[
            pl.BlockSpec((H, SB, DH), lambda i: (0, i, 0)),
            pl.BlockSpec((H, DH, D), lambda i: (0, 0, 0)),
        ],
        out_specs=pl.BlockSpec((SB, D), lambda i: (i, 0)),
        out_shape=jax.ShapeDtypeStruct((S, D), jnp.float32),
    )(attn_out, woT)

    return out[None]
